# baseline (speedup 1.0000x reference)
import functools

import jax
import jax.numpy as jnp
from jax import lax
from jax.experimental import pallas as pl
from jax.experimental.pallas import tpu as pltpu
from jax.experimental.pallas import tpu_sc as plsc

MEM_ROWS = 1_000_000
FDIM = 32
FPAD = 128
BATCH = 16384

NUM_CORES = 2
NUM_SUBCORES = 16
LANES = 16
NW = NUM_CORES * NUM_SUBCORES
ROWS_BASE = 31248
ROWS_LAST = MEM_ROWS - (NW - 1) * ROWS_BASE
NVREG = BATCH // LANES
CAP = BATCH + LANES
VCH = 2048


def _scatter_body(mem_ref, valid_hbm, feat_hbm, idx_hbm, valid_out,
                  idx_all, sel_idx, sel_pos, stamp, feat_buf, vbuf,
                  sem_in, sem_g, sem_s, sem_v):
    wid = lax.axis_index("s") * NUM_CORES + lax.axis_index("c")
    lo = wid * ROWS_BASE
    rows_w = jnp.where(wid == NW - 1, ROWS_LAST, ROWS_BASE)

    cp_idx = pltpu.async_copy(idx_hbm, idx_all, sem_in)

    def clr_body(i, carry):
        stamp[pl.ds(i * LANES, LANES)] = jnp.full((LANES,), -1, jnp.int32)
        return carry

    lax.fori_loop(0, ROWS_LAST // LANES, clr_body, jnp.int32(0))
    cp_idx.wait()

    lane = lax.iota(jnp.int32, LANES)

    def comp_body(i, off):
        v = idx_all[pl.ds(i * LANES, LANES)]
        pos = lane + i * LANES
        m = (v >= lo) & (v < lo + rows_w)
        mi = m.astype(jnp.int32)
        dst = off + plsc.cumsum(mi) - mi
        plsc.store_scatter(sel_idx, [dst], v, mask=m)
        plsc.store_scatter(sel_pos, [dst], pos, mask=m)
        return off + jnp.sum(mi)

    count = lax.fori_loop(0, NVREG, comp_body, jnp.int32(0))

    @pl.when(count > 0)
    def _():
        def stamp_body(i, carry):
            linear = lane + i * LANES
            vl = linear < count
            v = sel_idx[pl.ds(i * LANES, LANES)] - lo
            for s in range(LANES):
                plsc.store_scatter(stamp, [v], linear, mask=vl & (lane == s))
            return carry

        ngrp_in = (count + LANES - 1) // LANES
        lax.fori_loop(0, ngrp_in, stamp_body, jnp.int32(0))

        def keep_body(i, foff):
            linear = lane + i * LANES
            valid_lane = linear < count
            v = sel_idx[pl.ds(i * LANES, LANES)]
            p = sel_pos[pl.ds(i * LANES, LANES)]
            g = plsc.load_gather(stamp, [v - lo], mask=valid_lane)
            keep = valid_lane & (g == linear)
            ki = keep.astype(jnp.int32)
            dst = foff + plsc.cumsum(ki) - ki
            plsc.store_scatter(sel_idx, [dst], v, mask=keep)
            plsc.store_scatter(sel_pos, [dst], p, mask=keep)
            return foff + jnp.sum(ki)

        fcount = lax.fori_loop(0, ngrp_in, keep_body, jnp.int32(0))

        pad_pos = jnp.full((LANES,), fcount - 1, jnp.int32)
        last_i_v = plsc.load_gather(sel_idx, [pad_pos])
        last_p_v = plsc.load_gather(sel_pos, [pad_pos])
        base16 = (fcount // LANES) * LANES
        tmask = (base16 + lane) >= fcount
        plsc.store_scatter(sel_idx, [base16 + lane], last_i_v, mask=tmask)
        plsc.store_scatter(sel_pos, [base16 + lane], last_p_v, mask=tmask)

        def grp_body(gi, carry):
            pv = sel_pos[pl.ds(gi * LANES, LANES)]
            iv = sel_idx[pl.ds(gi * LANES, LANES)]
            for l in range(LANES):
                pltpu.async_copy(feat_hbm.at[pl.ds(pv[l], 1), :],
                                 mem_ref.at[pl.ds(iv[l], 1), :], sem_s)
            for l in range(LANES):
                pltpu.make_async_copy(feat_hbm.at[pl.ds(pv[l], 1), :],
                                      mem_ref.at[pl.ds(iv[l], 1), :],
                                      sem_s).wait()
            return carry

        ngrp = (fcount + LANES - 1) // LANES
        lax.fori_loop(0, ngrp, grp_body, jnp.int32(0))

    def vch_body(ci, off):
        base = pl.multiple_of(lo + off, 8)
        pltpu.async_copy(valid_hbm.at[pl.ds(base, VCH)], vbuf, sem_v).wait()

        def or_body(k, carry):
            o = vbuf[pl.ds(k * LANES, LANES)]
            st = stamp[pl.ds(pl.multiple_of(off + k * LANES, 8), LANES)]
            vbuf[pl.ds(k * LANES, LANES)] = o | (st >= 0).astype(jnp.int32)
            return carry

        lax.fori_loop(0, VCH // LANES, or_body, jnp.int32(0))
        pltpu.async_copy(vbuf, valid_out.at[pl.ds(base, VCH)], sem_v).wait()
        return off + VCH

    lax.fori_loop(0, rows_w // VCH, vch_body, jnp.int32(0))
    vch_body(jnp.int32(0), rows_w - VCH)


@functools.cache
def _sc_scatter():
    return pl.kernel(
        _scatter_body,
        out_type=jax.ShapeDtypeStruct((MEM_ROWS,), jnp.int32),
        mesh=plsc.VectorSubcoreMesh(
            core_axis_name="c", subcore_axis_name="s",
            num_cores=NUM_CORES, num_subcores=NUM_SUBCORES),
        compiler_params=pltpu.CompilerParams(needs_layout_passes=False),
        scratch_types=[
            pltpu.VMEM((BATCH,), jnp.int32),
            pltpu.VMEM((CAP,), jnp.int32),
            pltpu.VMEM((CAP,), jnp.int32),
            pltpu.VMEM((ROWS_LAST,), jnp.int32),
            pltpu.VMEM((LANES, FDIM), jnp.float32),
            pltpu.VMEM((VCH,), jnp.int32),
            pltpu.SemaphoreType.DMA,
            pltpu.SemaphoreType.DMA,
            pltpu.SemaphoreType.DMA,
            pltpu.SemaphoreType.DMA,
        ],
    )


def kernel(memory, memory_valid, features, indices):
    valid32 = memory_valid.astype(jnp.int32)
    mem_ref = jax.new_ref(memory)
    valid_new = _sc_scatter()(mem_ref, valid32, features, indices)
    return jax.freeze(mem_ref), (valid_new != 0)


# speedup vs baseline: 2.5864x; 2.5864x over previous
import functools

import jax
import jax.numpy as jnp
from jax import lax
from jax.experimental import pallas as pl
from jax.experimental.pallas import tpu as pltpu
from jax.experimental.pallas import tpu_sc as plsc

MEM_ROWS = 1_000_000
FDIM = 32
FPAD = 128
BATCH = 16384

NUM_CORES = 2
NUM_SUBCORES = 16
LANES = 16
NW = NUM_CORES * NUM_SUBCORES
ROWS_BASE = 31232
ROWS_LAST = MEM_ROWS - (NW - 1) * ROWS_BASE
BLK = 512
TAIL = ROWS_LAST % BLK
TAIL_OFF = MEM_ROWS - TAIL
NVREG = BATCH // LANES
CAP = BATCH + LANES
BCAP = BLK + LANES
VCH = 2048


def _scatter_body(mem_t, valid_hbm, feat_hbm, idx_hbm, out_t, valid_out,
                  idx_all, sel_idx, sel_pos, stamp, blk_c, blk_p,
                  feat_grp, buf, tbuf, vbuf,
                  sem_in, sem_g, sem_b, sem_v):
    wid = lax.axis_index("s") * NUM_CORES + lax.axis_index("c")
    lo = wid * ROWS_BASE
    rows_w = jnp.where(wid == NW - 1, ROWS_LAST, ROWS_BASE)

    cp_idx = pltpu.async_copy(idx_hbm, idx_all, sem_in)

    def clr_body(i, carry):
        stamp[pl.ds(i * LANES, LANES)] = jnp.full((LANES,), -1, jnp.int32)
        return carry

    lax.fori_loop(0, ROWS_LAST // LANES, clr_body, jnp.int32(0))
    cp_idx.wait()

    lane = lax.iota(jnp.int32, LANES)

    def comp_body(i, off):
        v = idx_all[pl.ds(i * LANES, LANES)]
        pos = lane + i * LANES
        m = (v >= lo) & (v < lo + rows_w)
        mi = m.astype(jnp.int32)
        dst = off + plsc.cumsum(mi) - mi
        plsc.store_scatter(sel_idx, [dst], v, mask=m)
        plsc.store_scatter(sel_pos, [dst], pos, mask=m)
        return off + jnp.sum(mi)

    count = lax.fori_loop(0, NVREG, comp_body, jnp.int32(0))

    def stamp_body(i, carry):
        linear = lane + i * LANES
        vl = linear < count
        v = sel_idx[pl.ds(i * LANES, LANES)] - lo
        for s in range(LANES):
            plsc.store_scatter(stamp, [v], linear, mask=vl & (lane == s))
        return carry

    ngrp_in = (count + LANES - 1) // LANES
    lax.fori_loop(0, ngrp_in, stamp_body, jnp.int32(0))

    def keep_body(i, foff):
        linear = lane + i * LANES
        valid_lane = linear < count
        v = sel_idx[pl.ds(i * LANES, LANES)]
        p = sel_pos[pl.ds(i * LANES, LANES)]
        g = plsc.load_gather(stamp, [v - lo], mask=valid_lane)
        keep = valid_lane & (g == linear)
        ki = keep.astype(jnp.int32)
        dst = foff + plsc.cumsum(ki) - ki
        plsc.store_scatter(sel_idx, [dst], v, mask=keep)
        plsc.store_scatter(sel_pos, [dst], p, mask=keep)
        return foff + jnp.sum(ki)

    fcount = lax.fori_loop(0, ngrp_in, keep_body, jnp.int32(0))

    @pl.when(fcount > 0)
    def _():
        pad_pos = jnp.full((LANES,), fcount - 1, jnp.int32)
        last_i_v = plsc.load_gather(sel_idx, [pad_pos])
        last_p_v = plsc.load_gather(sel_pos, [pad_pos])
        base16 = (fcount // LANES) * LANES
        tmask = (base16 + lane) >= fcount
        plsc.store_scatter(sel_idx, [base16 + lane], last_i_v, mask=tmask)
        plsc.store_scatter(sel_pos, [base16 + lane], last_p_v, mask=tmask)

    ngrp_sel = (fcount + LANES - 1) // LANES

    def update_panel(panel, bbase, width):
        def scan_body(i, bcnt):
            linear = lane + i * LANES
            vl = linear < fcount
            v = sel_idx[pl.ds(i * LANES, LANES)]
            p = sel_pos[pl.ds(i * LANES, LANES)]
            m = vl & (v >= bbase) & (v < bbase + width)
            mi = m.astype(jnp.int32)
            dst = bcnt + plsc.cumsum(mi) - mi
            plsc.store_scatter(blk_c, [dst], v - bbase, mask=m)
            plsc.store_scatter(blk_p, [dst], p, mask=m)
            return bcnt + jnp.sum(mi)

        bcnt = lax.fori_loop(0, ngrp_sel, scan_body, jnp.int32(0))

        @pl.when(bcnt > 0)
        def _():
            bpad = jnp.full((LANES,), bcnt - 1, jnp.int32)
            lc = plsc.load_gather(blk_c, [bpad])
            lp = plsc.load_gather(blk_p, [bpad])
            b16 = (bcnt // LANES) * LANES
            tm = (b16 + lane) >= bcnt
            plsc.store_scatter(blk_c, [b16 + lane], lc, mask=tm)
            plsc.store_scatter(blk_p, [b16 + lane], lp, mask=tm)

            def grp_body(g, carry):
                pv = blk_p[pl.ds(g * LANES, LANES)]
                cv = blk_c[pl.ds(g * LANES, LANES)]
                pltpu.async_copy(feat_hbm.at[pv, :], feat_grp, sem_g).wait()
                for l in range(LANES):
                    cl = cv[l]
                    for h in range(FDIM // LANES):
                        val = feat_grp[l, pl.ds(h * LANES, LANES)]
                        plsc.store_scatter(
                            panel, [lane + h * LANES,
                                    jnp.full((LANES,), cl, jnp.int32)], val)
                return carry

            bgrp = (bcnt + LANES - 1) // LANES
            lax.fori_loop(0, bgrp, grp_body, jnp.int32(0))

    def blk_body(b, carry):
        bbase = pl.multiple_of(lo + b * BLK, 128)
        pltpu.async_copy(mem_t.at[:, pl.ds(bbase, BLK)], buf, sem_b).wait()
        update_panel(buf, bbase, BLK)
        pltpu.async_copy(buf, out_t.at[:, pl.ds(bbase, BLK)], sem_b).wait()
        return carry

    lax.fori_loop(0, rows_w // BLK, blk_body, jnp.int32(0))

    @pl.when(wid == NW - 1)
    def _():
        pltpu.async_copy(mem_t.at[:, pl.ds(TAIL_OFF, TAIL)], tbuf,
                         sem_b).wait()
        update_panel(tbuf, jnp.int32(TAIL_OFF), TAIL)
        pltpu.async_copy(tbuf, out_t.at[:, pl.ds(TAIL_OFF, TAIL)],
                         sem_b).wait()

    def vch_body(ci, off):
        base = pl.multiple_of(lo + off, 8)
        pltpu.async_copy(valid_hbm.at[pl.ds(base, VCH)], vbuf, sem_v).wait()

        def or_body(k, carry):
            o = vbuf[pl.ds(k * LANES, LANES)]
            st = stamp[pl.ds(pl.multiple_of(off + k * LANES, 8), LANES)]
            vbuf[pl.ds(k * LANES, LANES)] = o | (st >= 0).astype(jnp.int32)
            return carry

        lax.fori_loop(0, VCH // LANES, or_body, jnp.int32(0))
        pltpu.async_copy(vbuf, valid_out.at[pl.ds(base, VCH)], sem_v).wait()
        return off + VCH

    lax.fori_loop(0, rows_w // VCH, vch_body, jnp.int32(0))
    vch_body(jnp.int32(0), rows_w - VCH)


@functools.cache
def _sc_scatter():
    return pl.kernel(
        _scatter_body,
        out_type=(jax.ShapeDtypeStruct((FDIM, MEM_ROWS), jnp.float32),
                  jax.ShapeDtypeStruct((MEM_ROWS,), jnp.int32)),
        mesh=plsc.VectorSubcoreMesh(
            core_axis_name="c", subcore_axis_name="s",
            num_cores=NUM_CORES, num_subcores=NUM_SUBCORES),
        compiler_params=pltpu.CompilerParams(needs_layout_passes=False),
        scratch_types=[
            pltpu.VMEM((BATCH,), jnp.int32),
            pltpu.VMEM((CAP,), jnp.int32),
            pltpu.VMEM((CAP,), jnp.int32),
            pltpu.VMEM((ROWS_LAST,), jnp.int32),
            pltpu.VMEM((BCAP,), jnp.int32),
            pltpu.VMEM((BCAP,), jnp.int32),
            pltpu.VMEM((LANES, FPAD), jnp.float32),
            pltpu.VMEM((FDIM, BLK), jnp.float32),
            pltpu.VMEM((FDIM, TAIL), jnp.float32),
            pltpu.VMEM((VCH,), jnp.int32),
            pltpu.SemaphoreType.DMA,
            pltpu.SemaphoreType.DMA,
            pltpu.SemaphoreType.DMA,
            pltpu.SemaphoreType.DMA,
        ],
    )


def kernel(memory, memory_valid, features, indices):
    valid32 = memory_valid.astype(jnp.int32)
    feats128 = jnp.pad(features, ((0, 0), (0, FPAD - FDIM)))
    out_t, valid_new = _sc_scatter()(memory.T, valid32, feats128, indices)
    return out_t.T, (valid_new != 0)


# speedup vs baseline: 3.5133x; 1.3584x over previous
import functools

import jax
import jax.numpy as jnp
from jax import lax
from jax.experimental import pallas as pl
from jax.experimental.pallas import tpu as pltpu
from jax.experimental.pallas import tpu_sc as plsc

MEM_ROWS = 1_000_000
FDIM = 32
FPAD = 128
BATCH = 16384

NUM_CORES = 2
NUM_SUBCORES = 16
LANES = 16
NW = NUM_CORES * NUM_SUBCORES
ROWS_BASE = 31232
ROWS_LAST = MEM_ROWS - (NW - 1) * ROWS_BASE
BLK = 512
TAIL = ROWS_LAST % BLK
TAIL_OFF = MEM_ROWS - TAIL
NVREG = BATCH // LANES
CAP = BATCH + LANES
BCAP = BLK + LANES
VCH = 2048


def _scatter_body(mem_t, valid_hbm, feat_hbm, idx_hbm, out_t, valid_out,
                  idx_all, sel_idx, sel_pos, stamp, blk_c, blk_p,
                  feat_grp, buf0, buf1, tbuf, vbuf,
                  sem_in, sem_g, sem_bi, sem_bo, sem_v):
    wid = lax.axis_index("s") * NUM_CORES + lax.axis_index("c")
    lo = wid * ROWS_BASE
    rows_w = jnp.where(wid == NW - 1, ROWS_LAST, ROWS_BASE)

    cp_idx = pltpu.async_copy(idx_hbm, idx_all, sem_in)

    def clr_body(i, carry):
        stamp[pl.ds(i * LANES, LANES)] = jnp.full((LANES,), -1, jnp.int32)
        return carry

    lax.fori_loop(0, ROWS_LAST // LANES, clr_body, jnp.int32(0), unroll=4)
    cp_idx.wait()

    lane = lax.iota(jnp.int32, LANES)

    def comp_body(i, off):
        v = idx_all[pl.ds(i * LANES, LANES)]
        pos = lane + i * LANES
        m = (v >= lo) & (v < lo + rows_w)
        mi = m.astype(jnp.int32)
        c = plsc.cumsum(mi)
        dst = off + c - mi
        plsc.store_scatter(sel_idx, [dst], v, mask=m)
        plsc.store_scatter(sel_pos, [dst], pos, mask=m)
        return off + c[LANES - 1]

    count = lax.fori_loop(0, NVREG, comp_body, jnp.int32(0))

    def stamp_body(i, carry):
        linear = lane + i * LANES
        vl = linear < count
        v = sel_idx[pl.ds(i * LANES, LANES)] - lo
        for s in range(LANES):
            plsc.store_scatter(stamp, [v], linear, mask=vl & (lane == s))
        return carry

    ngrp_in = (count + LANES - 1) // LANES
    lax.fori_loop(0, ngrp_in, stamp_body, jnp.int32(0))

    def keep_body(i, foff):
        linear = lane + i * LANES
        valid_lane = linear < count
        v = sel_idx[pl.ds(i * LANES, LANES)]
        p = sel_pos[pl.ds(i * LANES, LANES)]
        g = plsc.load_gather(stamp, [v - lo], mask=valid_lane)
        keep = valid_lane & (g == linear)
        ki = keep.astype(jnp.int32)
        ck = plsc.cumsum(ki)
        dst = foff + ck - ki
        plsc.store_scatter(sel_idx, [dst], v, mask=keep)
        plsc.store_scatter(sel_pos, [dst], p, mask=keep)
        return foff + ck[LANES - 1]

    fcount = lax.fori_loop(0, ngrp_in, keep_body, jnp.int32(0))

    @pl.when(fcount > 0)
    def _():
        pad_pos = jnp.full((LANES,), fcount - 1, jnp.int32)
        last_i_v = plsc.load_gather(sel_idx, [pad_pos])
        last_p_v = plsc.load_gather(sel_pos, [pad_pos])
        base16 = (fcount // LANES) * LANES
        tmask = (base16 + lane) >= fcount
        plsc.store_scatter(sel_idx, [base16 + lane], last_i_v, mask=tmask)
        plsc.store_scatter(sel_pos, [base16 + lane], last_p_v, mask=tmask)

    ngrp_sel = (fcount + LANES - 1) // LANES

    def update_panel(panel, bbase, width):
        def scan_body(i, bcnt):
            linear = lane + i * LANES
            vl = linear < fcount
            v = sel_idx[pl.ds(i * LANES, LANES)]
            p = sel_pos[pl.ds(i * LANES, LANES)]
            m = vl & (v >= bbase) & (v < bbase + width)
            mi = m.astype(jnp.int32)
            cb = plsc.cumsum(mi)
            dst = bcnt + cb - mi
            plsc.store_scatter(blk_c, [dst], v - bbase, mask=m)
            plsc.store_scatter(blk_p, [dst], p, mask=m)
            return bcnt + cb[LANES - 1]

        bcnt = lax.fori_loop(0, ngrp_sel, scan_body, jnp.int32(0))

        @pl.when(bcnt > 0)
        def _():
            bpad = jnp.full((LANES,), bcnt - 1, jnp.int32)
            lc = plsc.load_gather(blk_c, [bpad])
            lp = plsc.load_gather(blk_p, [bpad])
            b16 = (bcnt // LANES) * LANES
            tm = (b16 + lane) >= bcnt
            plsc.store_scatter(blk_c, [b16 + lane], lc, mask=tm)
            plsc.store_scatter(blk_p, [b16 + lane], lp, mask=tm)

            def grp_body(g, carry):
                pv = blk_p[pl.ds(g * LANES, LANES)]
                cv = blk_c[pl.ds(g * LANES, LANES)]
                pltpu.async_copy(feat_hbm.at[pv, :], feat_grp, sem_g).wait()
                for l in range(LANES):
                    cl = cv[l]
                    for h in range(FDIM // LANES):
                        val = feat_grp[l, pl.ds(h * LANES, LANES)]
                        plsc.store_scatter(
                            panel, [lane + h * LANES,
                                    jnp.full((LANES,), cl, jnp.int32)], val)
                return carry

            bgrp = (bcnt + LANES - 1) // LANES
            lax.fori_loop(0, bgrp, grp_body, jnp.int32(0))

    nblk = rows_w // BLK

    def start_in(b, dstbuf):
        bbase = pl.multiple_of(lo + b * BLK, 128)
        pltpu.async_copy(mem_t.at[:, pl.ds(bbase, BLK)], dstbuf, sem_bi)

    def wait_in():
        pltpu.make_async_copy(mem_t.at[:, pl.ds(0, BLK)], buf0, sem_bi).wait()

    def start_out(b, srcbuf):
        bbase = pl.multiple_of(lo + b * BLK, 128)
        pltpu.async_copy(srcbuf, out_t.at[:, pl.ds(bbase, BLK)], sem_bo)

    def wait_out():
        pltpu.make_async_copy(buf0, out_t.at[:, pl.ds(0, BLK)], sem_bo).wait()

    start_in(jnp.int32(0), buf0)

    def blk_body(b, carry):
        def step(cur, nxt):
            @pl.when(b + 1 < nblk)
            def _():
                @pl.when(b >= 1)
                def _():
                    wait_out()
                start_in(b + 1, nxt)

            wait_in()
            bbase = pl.multiple_of(lo + b * BLK, 128)
            update_panel(cur, bbase, BLK)
            start_out(b, cur)

        @pl.when(b % 2 == 0)
        def _():
            step(buf0, buf1)

        @pl.when(b % 2 == 1)
        def _():
            step(buf1, buf0)

        return carry

    lax.fori_loop(0, nblk, blk_body, jnp.int32(0))

    @pl.when(nblk >= 2)
    def _():
        wait_out()

    wait_out()

    @pl.when(wid == NW - 1)
    def _():
        pltpu.async_copy(mem_t.at[:, pl.ds(TAIL_OFF, TAIL)], tbuf,
                         sem_bi).wait()
        update_panel(tbuf, jnp.int32(TAIL_OFF), TAIL)
        pltpu.async_copy(tbuf, out_t.at[:, pl.ds(TAIL_OFF, TAIL)],
                         sem_bo).wait()

    def vch_body(ci, off):
        base = pl.multiple_of(lo + off, 8)
        pltpu.async_copy(valid_hbm.at[pl.ds(base, VCH)], vbuf, sem_v).wait()

        def or_body(k, carry):
            o = vbuf[pl.ds(k * LANES, LANES)]
            st = stamp[pl.ds(pl.multiple_of(off + k * LANES, 8), LANES)]
            vbuf[pl.ds(k * LANES, LANES)] = o | (st >= 0).astype(jnp.int32)
            return carry

        lax.fori_loop(0, VCH // LANES, or_body, jnp.int32(0), unroll=4)
        pltpu.async_copy(vbuf, valid_out.at[pl.ds(base, VCH)], sem_v).wait()
        return off + VCH

    lax.fori_loop(0, rows_w // VCH, vch_body, jnp.int32(0))
    vch_body(jnp.int32(0), rows_w - VCH)


@functools.cache
def _sc_scatter():
    return pl.kernel(
        _scatter_body,
        out_type=(jax.ShapeDtypeStruct((FDIM, MEM_ROWS), jnp.float32),
                  jax.ShapeDtypeStruct((MEM_ROWS,), jnp.int32)),
        mesh=plsc.VectorSubcoreMesh(
            core_axis_name="c", subcore_axis_name="s",
            num_cores=NUM_CORES, num_subcores=NUM_SUBCORES),
        compiler_params=pltpu.CompilerParams(needs_layout_passes=False),
        scratch_types=[
            pltpu.VMEM((BATCH,), jnp.int32),
            pltpu.VMEM((CAP,), jnp.int32),
            pltpu.VMEM((CAP,), jnp.int32),
            pltpu.VMEM((ROWS_LAST,), jnp.int32),
            pltpu.VMEM((BCAP,), jnp.int32),
            pltpu.VMEM((BCAP,), jnp.int32),
            pltpu.VMEM((LANES, FPAD), jnp.float32),
            pltpu.VMEM((FDIM, BLK), jnp.float32),
            pltpu.VMEM((FDIM, BLK), jnp.float32),
            pltpu.VMEM((FDIM, TAIL), jnp.float32),
            pltpu.VMEM((VCH,), jnp.int32),
            pltpu.SemaphoreType.DMA,
            pltpu.SemaphoreType.DMA,
            pltpu.SemaphoreType.DMA,
            pltpu.SemaphoreType.DMA,
            pltpu.SemaphoreType.DMA,
        ],
    )


def kernel(memory, memory_valid, features, indices):
    valid32 = memory_valid.astype(jnp.int32)
    feats128 = jnp.pad(features, ((0, 0), (0, FPAD - FDIM)))
    out_t, valid_new = _sc_scatter()(memory.T, valid32, feats128, indices)
    return out_t.T, (valid_new != 0)


# speedup vs baseline: 3.7656x; 1.0718x over previous
import functools

import jax
import jax.numpy as jnp
from jax import lax
from jax.experimental import pallas as pl
from jax.experimental.pallas import tpu as pltpu
from jax.experimental.pallas import tpu_sc as plsc

MEM_ROWS = 1_000_000
FDIM = 32
FPAD = 128
BATCH = 16384

NUM_CORES = 2
NUM_SUBCORES = 16
LANES = 16
NW = NUM_CORES * NUM_SUBCORES
ROWS_BASE = 31232
ROWS_LAST = MEM_ROWS - (NW - 1) * ROWS_BASE
BLK = 512
TAIL = ROWS_LAST % BLK
TAIL_OFF = MEM_ROWS - TAIL
NVREG = BATCH // LANES
CAP = BATCH + LANES
BCAP = BLK + LANES
VCH = 2048


def _scatter_body(mem_t, valid_hbm, feat_hbm, idx_hbm, out_t, valid_out,
                  idx_all, sel_idx, sel_pos, stamp, blk_c, blk_p,
                  feat_grp, buf0, buf1, tbuf, vbuf0, vbuf1, tvb,
                  sem_in, sem_g, sem_bi, sem_bo, sem_vi, sem_vo):
    wid = lax.axis_index("s") * NUM_CORES + lax.axis_index("c")
    lo = wid * ROWS_BASE
    rows_w = jnp.where(wid == NW - 1, ROWS_LAST, ROWS_BASE)

    cp_idx = pltpu.async_copy(idx_hbm, idx_all, sem_in)

    def clr_body(i, carry):
        stamp[pl.ds(i * LANES, LANES)] = jnp.full((LANES,), -1, jnp.int32)
        return carry

    lax.fori_loop(0, ROWS_LAST // LANES, clr_body, jnp.int32(0), unroll=4)
    cp_idx.wait()

    lane = lax.iota(jnp.int32, LANES)

    def comp_body(i, off):
        v = idx_all[pl.ds(i * LANES, LANES)]
        pos = lane + i * LANES
        m = (v >= lo) & (v < lo + rows_w)
        mi = m.astype(jnp.int32)
        c = plsc.cumsum(mi)
        dst = off + c - mi
        plsc.store_scatter(sel_idx, [dst], v, mask=m)
        plsc.store_scatter(sel_pos, [dst], pos, mask=m)
        return off + c[LANES - 1]

    count = lax.fori_loop(0, NVREG, comp_body, jnp.int32(0), unroll=2)

    def stamp_body(i, carry):
        linear = lane + i * LANES
        vl = linear < count
        v = sel_idx[pl.ds(i * LANES, LANES)] - lo
        for s in range(LANES):
            plsc.store_scatter(stamp, [v], linear, mask=vl & (lane == s))
        return carry

    ngrp_in = (count + LANES - 1) // LANES
    lax.fori_loop(0, ngrp_in, stamp_body, jnp.int32(0))

    def keep_body(i, foff):
        linear = lane + i * LANES
        valid_lane = linear < count
        v = sel_idx[pl.ds(i * LANES, LANES)]
        p = sel_pos[pl.ds(i * LANES, LANES)]
        g = plsc.load_gather(stamp, [v - lo], mask=valid_lane)
        keep = valid_lane & (g == linear)
        ki = keep.astype(jnp.int32)
        ck = plsc.cumsum(ki)
        dst = foff + ck - ki
        plsc.store_scatter(sel_idx, [dst], v, mask=keep)
        plsc.store_scatter(sel_pos, [dst], p, mask=keep)
        return foff + ck[LANES - 1]

    fcount = lax.fori_loop(0, ngrp_in, keep_body, jnp.int32(0))

    @pl.when(fcount > 0)
    def _():
        pad_pos = jnp.full((LANES,), fcount - 1, jnp.int32)
        last_i_v = plsc.load_gather(sel_idx, [pad_pos])
        last_p_v = plsc.load_gather(sel_pos, [pad_pos])
        base16 = (fcount // LANES) * LANES
        tmask = (base16 + lane) >= fcount
        plsc.store_scatter(sel_idx, [base16 + lane], last_i_v, mask=tmask)
        plsc.store_scatter(sel_pos, [base16 + lane], last_p_v, mask=tmask)

    ngrp_sel = (fcount + LANES - 1) // LANES

    def update_panel(panel, vpanel, bbase, width):
        def or_body(k, carry):
            o = vpanel[pl.ds(k * LANES, LANES)]
            st = stamp[pl.ds(pl.multiple_of(bbase - lo + k * LANES, 8),
                             LANES)]
            vpanel[pl.ds(k * LANES, LANES)] = (
                o | (st >= 0).astype(jnp.int32))
            return carry

        lax.fori_loop(0, width // LANES, or_body, jnp.int32(0), unroll=4)
        def scan_body(i, bcnt):
            linear = lane + i * LANES
            vl = linear < fcount
            v = sel_idx[pl.ds(i * LANES, LANES)]
            p = sel_pos[pl.ds(i * LANES, LANES)]
            m = vl & (v >= bbase) & (v < bbase + width)
            mi = m.astype(jnp.int32)
            cb = plsc.cumsum(mi)
            dst = bcnt + cb - mi
            plsc.store_scatter(blk_c, [dst], v - bbase, mask=m)
            plsc.store_scatter(blk_p, [dst], p, mask=m)
            return bcnt + cb[LANES - 1]

        bcnt = lax.fori_loop(0, ngrp_sel, scan_body, jnp.int32(0))

        @pl.when(bcnt > 0)
        def _():
            bpad = jnp.full((LANES,), bcnt - 1, jnp.int32)
            lc = plsc.load_gather(blk_c, [bpad])
            lp = plsc.load_gather(blk_p, [bpad])
            b16 = (bcnt // LANES) * LANES
            tm = (b16 + lane) >= bcnt
            plsc.store_scatter(blk_c, [b16 + lane], lc, mask=tm)
            plsc.store_scatter(blk_p, [b16 + lane], lp, mask=tm)

            def grp_body(g, carry):
                pv = blk_p[pl.ds(g * LANES, LANES)]
                cv = blk_c[pl.ds(g * LANES, LANES)]
                pltpu.async_copy(feat_hbm.at[pv, :], feat_grp, sem_g).wait()
                for l in range(LANES):
                    cl = cv[l]
                    for h in range(FDIM // LANES):
                        val = feat_grp[l, pl.ds(h * LANES, LANES)]
                        plsc.store_scatter(
                            panel, [lane + h * LANES,
                                    jnp.full((LANES,), cl, jnp.int32)], val)
                return carry

            bgrp = (bcnt + LANES - 1) // LANES
            lax.fori_loop(0, bgrp, grp_body, jnp.int32(0))

    nblk = rows_w // BLK

    def start_in(b, dstbuf, vdst):
        bbase = pl.multiple_of(lo + b * BLK, 128)
        pltpu.async_copy(mem_t.at[:, pl.ds(bbase, BLK)], dstbuf, sem_bi)
        pltpu.async_copy(valid_hbm.at[pl.ds(bbase, BLK)], vdst, sem_vi)

    def wait_in():
        pltpu.make_async_copy(mem_t.at[:, pl.ds(0, BLK)], buf0, sem_bi).wait()
        pltpu.make_async_copy(valid_hbm.at[pl.ds(0, BLK)], vbuf0,
                              sem_vi).wait()

    def start_out(b, srcbuf, vsrc):
        bbase = pl.multiple_of(lo + b * BLK, 128)
        pltpu.async_copy(srcbuf, out_t.at[:, pl.ds(bbase, BLK)], sem_bo)
        pltpu.async_copy(vsrc, valid_out.at[pl.ds(bbase, BLK)], sem_vo)

    def wait_out():
        pltpu.make_async_copy(buf0, out_t.at[:, pl.ds(0, BLK)], sem_bo).wait()
        pltpu.make_async_copy(vbuf0, valid_out.at[pl.ds(0, BLK)],
                              sem_vo).wait()

    start_in(jnp.int32(0), buf0, vbuf0)

    def blk_body(b, carry):
        def step(cur, nxt, vcur, vnxt):
            @pl.when(b + 1 < nblk)
            def _():
                @pl.when(b >= 1)
                def _():
                    wait_out()
                start_in(b + 1, nxt, vnxt)

            wait_in()
            bbase = pl.multiple_of(lo + b * BLK, 128)
            update_panel(cur, vcur, bbase, BLK)
            start_out(b, cur, vcur)

        @pl.when(b % 2 == 0)
        def _():
            step(buf0, buf1, vbuf0, vbuf1)

        @pl.when(b % 2 == 1)
        def _():
            step(buf1, buf0, vbuf1, vbuf0)

        return carry

    lax.fori_loop(0, nblk, blk_body, jnp.int32(0))

    @pl.when(nblk >= 2)
    def _():
        wait_out()

    wait_out()

    @pl.when(wid == NW - 1)
    def _():
        pltpu.async_copy(mem_t.at[:, pl.ds(TAIL_OFF, TAIL)], tbuf, sem_bi)
        pltpu.async_copy(valid_hbm.at[pl.ds(TAIL_OFF, TAIL)], tvb,
                         sem_vi).wait()
        pltpu.make_async_copy(mem_t.at[:, pl.ds(TAIL_OFF, TAIL)], tbuf,
                              sem_bi).wait()
        update_panel(tbuf, tvb, jnp.int32(TAIL_OFF), TAIL)
        pltpu.async_copy(tbuf, out_t.at[:, pl.ds(TAIL_OFF, TAIL)], sem_bo)
        pltpu.async_copy(tvb, valid_out.at[pl.ds(TAIL_OFF, TAIL)],
                         sem_vo).wait()
        pltpu.make_async_copy(tbuf, out_t.at[:, pl.ds(TAIL_OFF, TAIL)],
                              sem_bo).wait()


@functools.cache
def _sc_scatter():
    return pl.kernel(
        _scatter_body,
        out_type=(jax.ShapeDtypeStruct((FDIM, MEM_ROWS), jnp.float32),
                  jax.ShapeDtypeStruct((MEM_ROWS,), jnp.int32)),
        mesh=plsc.VectorSubcoreMesh(
            core_axis_name="c", subcore_axis_name="s",
            num_cores=NUM_CORES, num_subcores=NUM_SUBCORES),
        compiler_params=pltpu.CompilerParams(needs_layout_passes=False),
        scratch_types=[
            pltpu.VMEM((BATCH,), jnp.int32),
            pltpu.VMEM((CAP,), jnp.int32),
            pltpu.VMEM((CAP,), jnp.int32),
            pltpu.VMEM((ROWS_LAST,), jnp.int32),
            pltpu.VMEM((BCAP,), jnp.int32),
            pltpu.VMEM((BCAP,), jnp.int32),
            pltpu.VMEM((LANES, FPAD), jnp.float32),
            pltpu.VMEM((FDIM, BLK), jnp.float32),
            pltpu.VMEM((FDIM, BLK), jnp.float32),
            pltpu.VMEM((FDIM, TAIL), jnp.float32),
            pltpu.VMEM((BLK,), jnp.int32),
            pltpu.VMEM((BLK,), jnp.int32),
            pltpu.VMEM((TAIL,), jnp.int32),
            pltpu.SemaphoreType.DMA,
            pltpu.SemaphoreType.DMA,
            pltpu.SemaphoreType.DMA,
            pltpu.SemaphoreType.DMA,
            pltpu.SemaphoreType.DMA,
            pltpu.SemaphoreType.DMA,
        ],
    )


def kernel(memory, memory_valid, features, indices):
    valid32 = memory_valid.astype(jnp.int32)
    feats128 = jnp.pad(features, ((0, 0), (0, FPAD - FDIM)))
    out_t, valid_new = _sc_scatter()(memory.T, valid32, feats128, indices)
    return out_t.T, (valid_new != 0)


# speedup vs baseline: 3.8374x; 1.0191x over previous
import functools

import jax
import jax.numpy as jnp
from jax import lax
from jax.experimental import pallas as pl
from jax.experimental.pallas import tpu as pltpu
from jax.experimental.pallas import tpu_sc as plsc

MEM_ROWS = 1_000_000
FDIM = 32
FPAD = 128
BATCH = 16384

NUM_CORES = 2
NUM_SUBCORES = 16
LANES = 16
NW = NUM_CORES * NUM_SUBCORES
ROWS_BASE = 31232
ROWS_LAST = MEM_ROWS - (NW - 1) * ROWS_BASE
BLK = 512
TAIL = ROWS_LAST % BLK
TAIL_OFF = MEM_ROWS - TAIL
NVREG = BATCH // LANES
CAP = BATCH + LANES
BCAP = BLK + LANES
VCH = 2048


def _scatter_body(mem_t, valid_hbm, feat_hbm, idx_hbm, out_t, valid_out,
                  idx_all, sel_idx, sel_pos, stamp, blk_c, blk_p,
                  feat_grp, buf0, buf1, tbuf, vbuf0, vbuf1, tvb,
                  sem_in, sem_g, sem_bi, sem_bo, sem_vi, sem_vo):
    wid = lax.axis_index("s") * NUM_CORES + lax.axis_index("c")
    lo = wid * ROWS_BASE
    rows_w = jnp.where(wid == NW - 1, ROWS_LAST, ROWS_BASE)

    cp_idx = pltpu.async_copy(idx_hbm, idx_all, sem_in)

    def clr_body(i, carry):
        stamp[pl.ds(i * LANES, LANES)] = jnp.full((LANES,), -1, jnp.int32)
        return carry

    lax.fori_loop(0, ROWS_LAST // LANES, clr_body, jnp.int32(0), unroll=4)
    cp_idx.wait()

    lane = lax.iota(jnp.int32, LANES)

    def comp_body(i, off):
        v = idx_all[pl.ds(i * LANES, LANES)]
        pos = lane + i * LANES
        m = (v >= lo) & (v < lo + rows_w)
        mi = m.astype(jnp.int32)
        c = plsc.cumsum(mi)
        dst = off + c - mi
        plsc.store_scatter(sel_idx, [dst], v, mask=m)
        plsc.store_scatter(sel_pos, [dst], pos, mask=m)
        return off + c[LANES - 1]

    count = lax.fori_loop(0, NVREG, comp_body, jnp.int32(0), unroll=2)

    def stamp_body(i, carry):
        linear = lane + i * LANES
        vl = linear < count
        v = sel_idx[pl.ds(i * LANES, LANES)] - lo
        for s in range(LANES):
            plsc.store_scatter(stamp, [v], linear, mask=vl & (lane == s))
        return carry

    ngrp_in = (count + LANES - 1) // LANES
    lax.fori_loop(0, ngrp_in, stamp_body, jnp.int32(0))

    def keep_body(i, foff):
        linear = lane + i * LANES
        valid_lane = linear < count
        v = sel_idx[pl.ds(i * LANES, LANES)]
        p = sel_pos[pl.ds(i * LANES, LANES)]
        g = plsc.load_gather(stamp, [v - lo], mask=valid_lane)
        keep = valid_lane & (g == linear)
        ki = keep.astype(jnp.int32)
        ck = plsc.cumsum(ki)
        dst = foff + ck - ki
        plsc.store_scatter(sel_idx, [dst], v, mask=keep)
        plsc.store_scatter(sel_pos, [dst], p, mask=keep)
        return foff + ck[LANES - 1]

    fcount = lax.fori_loop(0, ngrp_in, keep_body, jnp.int32(0))

    @pl.when(fcount > 0)
    def _():
        pad_pos = jnp.full((LANES,), fcount - 1, jnp.int32)
        last_i_v = plsc.load_gather(sel_idx, [pad_pos])
        last_p_v = plsc.load_gather(sel_pos, [pad_pos])
        base16 = (fcount // LANES) * LANES
        tmask = (base16 + lane) >= fcount
        plsc.store_scatter(sel_idx, [base16 + lane], last_i_v, mask=tmask)
        plsc.store_scatter(sel_pos, [base16 + lane], last_p_v, mask=tmask)

    ngrp_sel = (fcount + LANES - 1) // LANES

    def update_panel(panel, vpanel, bbase, width):
        def scan_body(i, bcnt):
            linear = lane + i * LANES
            vl = linear < fcount
            v = sel_idx[pl.ds(i * LANES, LANES)]
            p = sel_pos[pl.ds(i * LANES, LANES)]
            m = vl & (v >= bbase) & (v < bbase + width)
            mi = m.astype(jnp.int32)
            cb = plsc.cumsum(mi)
            dst = bcnt + cb - mi
            plsc.store_scatter(blk_c, [dst], v - bbase, mask=m)
            plsc.store_scatter(blk_p, [dst], p, mask=m)
            return bcnt + cb[LANES - 1]

        bcnt = lax.fori_loop(0, ngrp_sel, scan_body, jnp.int32(0))

        @pl.when(bcnt > 0)
        def _():
            bpad = jnp.full((LANES,), bcnt - 1, jnp.int32)
            lc = plsc.load_gather(blk_c, [bpad])
            lp = plsc.load_gather(blk_p, [bpad])
            b16 = (bcnt // LANES) * LANES
            tm = (b16 + lane) >= bcnt
            plsc.store_scatter(blk_c, [b16 + lane], lc, mask=tm)
            plsc.store_scatter(blk_p, [b16 + lane], lp, mask=tm)

            ones16 = jnp.ones((LANES,), jnp.int32)

            def grp_body(g, carry):
                pv = blk_p[pl.ds(g * LANES, LANES)]
                cv = blk_c[pl.ds(g * LANES, LANES)]
                plsc.store_scatter(vpanel, [cv], ones16)
                pltpu.async_copy(feat_hbm.at[pv, :], feat_grp, sem_g).wait()
                for l in range(LANES):
                    cl = cv[l]
                    for h in range(FDIM // LANES):
                        val = feat_grp[l, pl.ds(h * LANES, LANES)]
                        plsc.store_scatter(
                            panel, [lane + h * LANES,
                                    jnp.full((LANES,), cl, jnp.int32)], val)
                return carry

            bgrp = (bcnt + LANES - 1) // LANES
            lax.fori_loop(0, bgrp, grp_body, jnp.int32(0))

    nblk = rows_w // BLK

    def start_in(b, dstbuf, vdst):
        bbase = pl.multiple_of(lo + b * BLK, 128)
        pltpu.async_copy(mem_t.at[:, pl.ds(bbase, BLK)], dstbuf, sem_bi)
        pltpu.async_copy(valid_hbm.at[pl.ds(bbase, BLK)], vdst, sem_vi)

    def wait_in():
        pltpu.make_async_copy(mem_t.at[:, pl.ds(0, BLK)], buf0, sem_bi).wait()
        pltpu.make_async_copy(valid_hbm.at[pl.ds(0, BLK)], vbuf0,
                              sem_vi).wait()

    def start_out(b, srcbuf, vsrc):
        bbase = pl.multiple_of(lo + b * BLK, 128)
        pltpu.async_copy(srcbuf, out_t.at[:, pl.ds(bbase, BLK)], sem_bo)
        pltpu.async_copy(vsrc, valid_out.at[pl.ds(bbase, BLK)], sem_vo)

    def wait_out():
        pltpu.make_async_copy(buf0, out_t.at[:, pl.ds(0, BLK)], sem_bo).wait()
        pltpu.make_async_copy(vbuf0, valid_out.at[pl.ds(0, BLK)],
                              sem_vo).wait()

    start_in(jnp.int32(0), buf0, vbuf0)

    def blk_body(b, carry):
        def step(cur, nxt, vcur, vnxt):
            @pl.when(b + 1 < nblk)
            def _():
                @pl.when(b >= 1)
                def _():
                    wait_out()
                start_in(b + 1, nxt, vnxt)

            wait_in()
            bbase = pl.multiple_of(lo + b * BLK, 128)
            update_panel(cur, vcur, bbase, BLK)
            start_out(b, cur, vcur)

        @pl.when(b % 2 == 0)
        def _():
            step(buf0, buf1, vbuf0, vbuf1)

        @pl.when(b % 2 == 1)
        def _():
            step(buf1, buf0, vbuf1, vbuf0)

        return carry

    lax.fori_loop(0, nblk, blk_body, jnp.int32(0))

    @pl.when(nblk >= 2)
    def _():
        wait_out()

    wait_out()

    @pl.when(wid == NW - 1)
    def _():
        pltpu.async_copy(mem_t.at[:, pl.ds(TAIL_OFF, TAIL)], tbuf, sem_bi)
        pltpu.async_copy(valid_hbm.at[pl.ds(TAIL_OFF, TAIL)], tvb,
                         sem_vi).wait()
        pltpu.make_async_copy(mem_t.at[:, pl.ds(TAIL_OFF, TAIL)], tbuf,
                              sem_bi).wait()
        update_panel(tbuf, tvb, jnp.int32(TAIL_OFF), TAIL)
        pltpu.async_copy(tbuf, out_t.at[:, pl.ds(TAIL_OFF, TAIL)], sem_bo)
        pltpu.async_copy(tvb, valid_out.at[pl.ds(TAIL_OFF, TAIL)],
                         sem_vo).wait()
        pltpu.make_async_copy(tbuf, out_t.at[:, pl.ds(TAIL_OFF, TAIL)],
                              sem_bo).wait()


@functools.cache
def _sc_scatter():
    return pl.kernel(
        _scatter_body,
        out_type=(jax.ShapeDtypeStruct((FDIM, MEM_ROWS), jnp.float32),
                  jax.ShapeDtypeStruct((MEM_ROWS,), jnp.int32)),
        mesh=plsc.VectorSubcoreMesh(
            core_axis_name="c", subcore_axis_name="s",
            num_cores=NUM_CORES, num_subcores=NUM_SUBCORES),
        compiler_params=pltpu.CompilerParams(needs_layout_passes=False),
        scratch_types=[
            pltpu.VMEM((BATCH,), jnp.int32),
            pltpu.VMEM((CAP,), jnp.int32),
            pltpu.VMEM((CAP,), jnp.int32),
            pltpu.VMEM((ROWS_LAST,), jnp.int32),
            pltpu.VMEM((BCAP,), jnp.int32),
            pltpu.VMEM((BCAP,), jnp.int32),
            pltpu.VMEM((LANES, FPAD), jnp.float32),
            pltpu.VMEM((FDIM, BLK), jnp.float32),
            pltpu.VMEM((FDIM, BLK), jnp.float32),
            pltpu.VMEM((FDIM, TAIL), jnp.float32),
            pltpu.VMEM((BLK,), jnp.int32),
            pltpu.VMEM((BLK,), jnp.int32),
            pltpu.VMEM((TAIL,), jnp.int32),
            pltpu.SemaphoreType.DMA,
            pltpu.SemaphoreType.DMA,
            pltpu.SemaphoreType.DMA,
            pltpu.SemaphoreType.DMA,
            pltpu.SemaphoreType.DMA,
            pltpu.SemaphoreType.DMA,
        ],
    )


def kernel(memory, memory_valid, features, indices):
    valid32 = memory_valid.astype(jnp.int32)
    feats128 = jnp.pad(features, ((0, 0), (0, FPAD - FDIM)))
    out_t, valid_new = _sc_scatter()(memory.T, valid32, feats128, indices)
    return out_t.T, (valid_new != 0)


# speedup vs baseline: 3.8639x; 1.0069x over previous
import functools

import jax
import jax.numpy as jnp
from jax import lax
from jax.experimental import pallas as pl
from jax.experimental.pallas import tpu as pltpu
from jax.experimental.pallas import tpu_sc as plsc

MEM_ROWS = 1_000_000
FDIM = 32
FPAD = 128
BATCH = 16384

NUM_CORES = 2
NUM_SUBCORES = 16
LANES = 16
NW = NUM_CORES * NUM_SUBCORES
ROWS_BASE = 31232
ROWS_LAST = MEM_ROWS - (NW - 1) * ROWS_BASE
BLK = 512
TAIL = ROWS_LAST % BLK
TAIL_OFF = MEM_ROWS - TAIL
NVREG = BATCH // LANES
CAP = BATCH + 2 * LANES
BCAP = BLK + LANES
VCH = 2048


def _scatter_body(mem_t, valid_hbm, feat_hbm, idx_hbm, out_t, valid_out,
                  idx_all, sel_idx, sel_pos, stamp, blk_c, blk_p,
                  feat_grp, buf0, buf1, tbuf, vbuf0, vbuf1, tvb,
                  sem_in, sem_g, sem_bi, sem_bo, sem_vi, sem_vo):
    wid = lax.axis_index("s") * NUM_CORES + lax.axis_index("c")
    lo = wid * ROWS_BASE
    rows_w = jnp.where(wid == NW - 1, ROWS_LAST, ROWS_BASE)

    cp_idx = pltpu.async_copy(idx_hbm, idx_all, sem_in)

    def clr_body(i, carry):
        stamp[pl.ds(i * LANES, LANES)] = jnp.full((LANES,), -1, jnp.int32)
        return carry

    lax.fori_loop(0, ROWS_LAST // LANES, clr_body, jnp.int32(0), unroll=4)
    cp_idx.wait()

    lane = lax.iota(jnp.int32, LANES)

    def comp_body(i, off):
        v = idx_all[pl.ds(i * LANES, LANES)]
        pos = lane + i * LANES
        m = (v >= lo) & (v < lo + rows_w)
        mi = m.astype(jnp.int32)
        c = plsc.cumsum(mi)
        dst = off + c - mi
        plsc.store_scatter(sel_idx, [dst], v, mask=m)
        plsc.store_scatter(sel_pos, [dst], pos, mask=m)
        return off + c[LANES - 1]

    count = lax.fori_loop(0, NVREG, comp_body, jnp.int32(0), unroll=2)

    def stamp_body(i, carry):
        linear = lane + i * LANES
        vl = linear < count
        v = sel_idx[pl.ds(i * LANES, LANES)] - lo
        for s in range(LANES):
            plsc.store_scatter(stamp, [v], linear, mask=vl & (lane == s))
        return carry

    ngrp_in = (count + LANES - 1) // LANES
    lax.fori_loop(0, ngrp_in, stamp_body, jnp.int32(0))

    def keep_body(i, foff):
        linear = lane + i * LANES
        valid_lane = linear < count
        v = sel_idx[pl.ds(i * LANES, LANES)]
        p = sel_pos[pl.ds(i * LANES, LANES)]
        g = plsc.load_gather(stamp, [v - lo], mask=valid_lane)
        keep = valid_lane & (g == linear)
        ki = keep.astype(jnp.int32)
        ck = plsc.cumsum(ki)
        dst = foff + ck - ki
        plsc.store_scatter(sel_idx, [dst], v, mask=keep)
        plsc.store_scatter(sel_pos, [dst], p, mask=keep)
        return foff + ck[LANES - 1]

    fcount = lax.fori_loop(0, ngrp_in, keep_body, jnp.int32(0))

    @pl.when(fcount > 0)
    def _():
        pad_pos = jnp.full((LANES,), fcount - 1, jnp.int32)
        last_i_v = plsc.load_gather(sel_idx, [pad_pos])
        last_p_v = plsc.load_gather(sel_pos, [pad_pos])
        base16 = (fcount // LANES) * LANES
        tmask = (base16 + lane) >= fcount
        plsc.store_scatter(sel_idx, [base16 + lane], last_i_v, mask=tmask)
        plsc.store_scatter(sel_pos, [base16 + lane], last_p_v, mask=tmask)

    ngrp_sel = (fcount + LANES - 1) // LANES

    def update_panel(panel, vpanel, bbase, width):
        def scan_body(i, bcnt):
            base = i * 2 * LANES
            l0 = lane + base
            l1 = l0 + LANES
            v0 = sel_idx[pl.ds(base, LANES)]
            v1 = sel_idx[pl.ds(base + LANES, LANES)]
            p0 = sel_pos[pl.ds(base, LANES)]
            p1 = sel_pos[pl.ds(base + LANES, LANES)]
            m0 = (l0 < fcount) & (v0 >= bbase) & (v0 < bbase + width)
            m1 = (l1 < fcount) & (v1 >= bbase) & (v1 < bbase + width)
            mi0 = m0.astype(jnp.int32)
            mi1 = m1.astype(jnp.int32)
            c0 = plsc.cumsum(mi0)
            c1 = plsc.cumsum(mi1)
            dst0 = bcnt + c0 - mi0
            n0 = bcnt + c0[LANES - 1]
            dst1 = n0 + c1 - mi1
            plsc.store_scatter(blk_c, [dst0], v0 - bbase, mask=m0)
            plsc.store_scatter(blk_p, [dst0], p0, mask=m0)
            plsc.store_scatter(blk_c, [dst1], v1 - bbase, mask=m1)
            plsc.store_scatter(blk_p, [dst1], p1, mask=m1)
            return n0 + c1[LANES - 1]

        bcnt = lax.fori_loop(0, (fcount + 2 * LANES - 1) // (2 * LANES),
                             scan_body, jnp.int32(0))

        @pl.when(bcnt > 0)
        def _():
            bpad = jnp.full((LANES,), bcnt - 1, jnp.int32)
            lc = plsc.load_gather(blk_c, [bpad])
            lp = plsc.load_gather(blk_p, [bpad])
            b16 = (bcnt // LANES) * LANES
            tm = (b16 + lane) >= bcnt
            plsc.store_scatter(blk_c, [b16 + lane], lc, mask=tm)
            plsc.store_scatter(blk_p, [b16 + lane], lp, mask=tm)

            ones16 = jnp.ones((LANES,), jnp.int32)

            def grp_body(g, carry):
                pv = blk_p[pl.ds(g * LANES, LANES)]
                cv = blk_c[pl.ds(g * LANES, LANES)]
                plsc.store_scatter(vpanel, [cv], ones16)
                pltpu.async_copy(feat_hbm.at[pv, :], feat_grp, sem_g).wait()
                for l in range(LANES):
                    cl = cv[l]
                    for h in range(FDIM // LANES):
                        val = feat_grp[l, pl.ds(h * LANES, LANES)]
                        plsc.store_scatter(
                            panel, [lane + h * LANES,
                                    jnp.full((LANES,), cl, jnp.int32)], val)
                return carry

            bgrp = (bcnt + LANES - 1) // LANES
            lax.fori_loop(0, bgrp, grp_body, jnp.int32(0))

    nblk = rows_w // BLK

    def start_in(b, dstbuf, vdst):
        bbase = pl.multiple_of(lo + b * BLK, 128)
        pltpu.async_copy(mem_t.at[:, pl.ds(bbase, BLK)], dstbuf, sem_bi)
        pltpu.async_copy(valid_hbm.at[pl.ds(bbase, BLK)], vdst, sem_vi)

    def wait_in():
        pltpu.make_async_copy(mem_t.at[:, pl.ds(0, BLK)], buf0, sem_bi).wait()
        pltpu.make_async_copy(valid_hbm.at[pl.ds(0, BLK)], vbuf0,
                              sem_vi).wait()

    def start_out(b, srcbuf, vsrc):
        bbase = pl.multiple_of(lo + b * BLK, 128)
        pltpu.async_copy(srcbuf, out_t.at[:, pl.ds(bbase, BLK)], sem_bo)
        pltpu.async_copy(vsrc, valid_out.at[pl.ds(bbase, BLK)], sem_vo)

    def wait_out():
        pltpu.make_async_copy(buf0, out_t.at[:, pl.ds(0, BLK)], sem_bo).wait()
        pltpu.make_async_copy(vbuf0, valid_out.at[pl.ds(0, BLK)],
                              sem_vo).wait()

    start_in(jnp.int32(0), buf0, vbuf0)

    def blk_body(b, carry):
        def step(cur, nxt, vcur, vnxt):
            @pl.when(b + 1 < nblk)
            def _():
                @pl.when(b >= 1)
                def _():
                    wait_out()
                start_in(b + 1, nxt, vnxt)

            wait_in()
            bbase = pl.multiple_of(lo + b * BLK, 128)
            update_panel(cur, vcur, bbase, BLK)
            start_out(b, cur, vcur)

        @pl.when(b % 2 == 0)
        def _():
            step(buf0, buf1, vbuf0, vbuf1)

        @pl.when(b % 2 == 1)
        def _():
            step(buf1, buf0, vbuf1, vbuf0)

        return carry

    lax.fori_loop(0, nblk, blk_body, jnp.int32(0))

    @pl.when(nblk >= 2)
    def _():
        wait_out()

    wait_out()

    @pl.when(wid == NW - 1)
    def _():
        pltpu.async_copy(mem_t.at[:, pl.ds(TAIL_OFF, TAIL)], tbuf, sem_bi)
        pltpu.async_copy(valid_hbm.at[pl.ds(TAIL_OFF, TAIL)], tvb,
                         sem_vi).wait()
        pltpu.make_async_copy(mem_t.at[:, pl.ds(TAIL_OFF, TAIL)], tbuf,
                              sem_bi).wait()
        update_panel(tbuf, tvb, jnp.int32(TAIL_OFF), TAIL)
        pltpu.async_copy(tbuf, out_t.at[:, pl.ds(TAIL_OFF, TAIL)], sem_bo)
        pltpu.async_copy(tvb, valid_out.at[pl.ds(TAIL_OFF, TAIL)],
                         sem_vo).wait()
        pltpu.make_async_copy(tbuf, out_t.at[:, pl.ds(TAIL_OFF, TAIL)],
                              sem_bo).wait()


@functools.cache
def _sc_scatter():
    return pl.kernel(
        _scatter_body,
        out_type=(jax.ShapeDtypeStruct((FDIM, MEM_ROWS), jnp.float32),
                  jax.ShapeDtypeStruct((MEM_ROWS,), jnp.int32)),
        mesh=plsc.VectorSubcoreMesh(
            core_axis_name="c", subcore_axis_name="s",
            num_cores=NUM_CORES, num_subcores=NUM_SUBCORES),
        compiler_params=pltpu.CompilerParams(needs_layout_passes=False),
        scratch_types=[
            pltpu.VMEM((BATCH,), jnp.int32),
            pltpu.VMEM((CAP,), jnp.int32),
            pltpu.VMEM((CAP,), jnp.int32),
            pltpu.VMEM((ROWS_LAST,), jnp.int32),
            pltpu.VMEM((BCAP,), jnp.int32),
            pltpu.VMEM((BCAP,), jnp.int32),
            pltpu.VMEM((LANES, FPAD), jnp.float32),
            pltpu.VMEM((FDIM, BLK), jnp.float32),
            pltpu.VMEM((FDIM, BLK), jnp.float32),
            pltpu.VMEM((FDIM, TAIL), jnp.float32),
            pltpu.VMEM((BLK,), jnp.int32),
            pltpu.VMEM((BLK,), jnp.int32),
            pltpu.VMEM((TAIL,), jnp.int32),
            pltpu.SemaphoreType.DMA,
            pltpu.SemaphoreType.DMA,
            pltpu.SemaphoreType.DMA,
            pltpu.SemaphoreType.DMA,
            pltpu.SemaphoreType.DMA,
            pltpu.SemaphoreType.DMA,
        ],
    )


def kernel(memory, memory_valid, features, indices):
    valid32 = memory_valid.astype(jnp.int32)
    feats128 = jnp.pad(features, ((0, 0), (0, FPAD - FDIM)))
    out_t, valid_new = _sc_scatter()(memory.T, valid32, feats128, indices)
    return out_t.T, (valid_new != 0)
